# baseline (speedup 1.0000x reference)
import functools

import jax
import jax.numpy as jnp
import numpy as np
from jax import lax
from jax.experimental import pallas as pl
from jax.experimental.pallas import tpu as pltpu
from jax.experimental.pallas import tpu_sc as plsc

LATENT = 128
IDX_CHUNK = 128
NBUF = 3

_j, _m = np.arange(LATENT // 2) // 16, np.arange(LATENT // 2) % 16
_PERM_LO = 32 * _j + _m
_PERM_HI = _PERM_LO + 16
_PAIR_PERM = np.empty(LATENT, dtype=np.int64)
_PAIR_PERM[0::2] = _PERM_LO
_PAIR_PERM[1::2] = _PERM_HI


@jax.jit
def _sc_gather(task_table, cancer_table, task_id, cancer_id):
    B = task_id.shape[0]
    D = task_table.shape[1]
    info = plsc.get_sparse_core_info()
    nw = info.num_cores * info.num_subcores
    b_per_w = B // nw
    n_chunk = b_per_w // IDX_CHUNK
    n_total = 2 * n_chunk

    mesh = plsc.VectorSubcoreMesh(core_axis_name="c", subcore_axis_name="s")

    @functools.partial(
        pl.kernel,
        mesh=mesh,
        compiler_params=pltpu.CompilerParams(needs_layout_passes=False),
        out_type=[
            jax.ShapeDtypeStruct((B, D // 2), jnp.float32),
            jax.ShapeDtypeStruct((B, D // 2), jnp.float32),
        ],
        scratch_types=[
            pltpu.VMEM((b_per_w,), jnp.int32),
            pltpu.VMEM((b_per_w,), jnp.int32),
            pltpu.VMEM((NBUF, IDX_CHUNK, D), jnp.float32),
            pltpu.VMEM((NBUF, IDX_CHUNK, D // 2), jnp.float32),
            pltpu.SemaphoreType.DMA((NBUF,)),
            pltpu.SemaphoreType.DMA((NBUF,)),
        ],
    )
    def gather2(t_tab, c_tab, t_idx, c_idx, t_out, c_out, tid_v, cid_v,
                rows_v, rows_bf, sem_g, sem_o):
        wid = lax.axis_index("s") * info.num_cores + lax.axis_index("c")
        base = wid * b_per_w
        pltpu.sync_copy(t_idx.at[pl.ds(base, b_per_w)], tid_v)
        pltpu.sync_copy(c_idx.at[pl.ds(base, b_per_w)], cid_v)

        def start_gather(c):
            tab = t_tab if c < n_chunk else c_tab
            idx_v = tid_v if c < n_chunk else cid_v
            j = c % n_chunk
            return pltpu.async_copy(
                tab.at[idx_v.at[pl.ds(j * IDX_CHUNK, IDX_CHUNK)]],
                rows_v.at[c % NBUF],
                sem_g.at[c % NBUF],
            )

        def pack_chunk(c):
            slot = c % NBUF

            def row8(i, carry):
                r = i * 8
                for rr in range(8):
                    for j in range(D // 32):
                        a = rows_v[slot, r + rr, pl.ds(32 * j, 16)]
                        bb = rows_v[slot, r + rr, pl.ds(32 * j + 16, 16)]
                        pk = plsc.pack(a, bb, format=plsc.PackFormat.INTERLEAVED)
                        rows_bf[slot, r + rr, pl.ds(16 * j, 16)] = plsc.bitcast(
                            pk, jnp.float32
                        )
                return carry

            lax.fori_loop(0, IDX_CHUNK // 8, row8, 0)

        def start_out(c):
            out = t_out if c < n_chunk else c_out
            j = c % n_chunk
            return pltpu.async_copy(
                rows_bf.at[c % NBUF],
                out.at[pl.ds(base + j * IDX_CHUNK, IDX_CHUNK)],
                sem_o.at[c % NBUF],
            )

        gcp = [None] * n_total
        ocp = [None] * n_total
        gcp[0] = start_gather(0)
        for c in range(n_total):
            if c + 1 < n_total:
                if c + 1 >= NBUF:
                    ocp[c + 1 - NBUF].wait()
                gcp[c + 1] = start_gather(c + 1)
            gcp[c].wait()
            pack_chunk(c)
            ocp[c] = start_out(c)
        for c in range(n_total - NBUF, n_total):
            ocp[c].wait()

    return gather2(task_table, cancer_table, task_id, cancer_id)


def _tc_project(te, ce, W1, W2, b2d):
    B = te.shape[0]
    BB = 8192

    def body(te_ref, ce_ref, w1_ref, w2_ref, b_ref, o_ref):
        acc = lax.dot_general(
            te_ref[...], w1_ref[...], (((1,), (1,)), ((), ())),
            preferred_element_type=jnp.float32,
        )
        acc += lax.dot_general(
            ce_ref[...], w2_ref[...], (((1,), (1,)), ((), ())),
            preferred_element_type=jnp.float32,
        )
        o_ref[...] = acc + b_ref[...]

    return pl.pallas_call(
        body,
        grid=(B // BB,),
        in_specs=[
            pl.BlockSpec((BB, LATENT), lambda i: (i, 0)),
            pl.BlockSpec((BB, LATENT), lambda i: (i, 0)),
            pl.BlockSpec((LATENT, LATENT), lambda i: (0, 0)),
            pl.BlockSpec((LATENT, LATENT), lambda i: (0, 0)),
            pl.BlockSpec((1, LATENT), lambda i: (0, 0)),
        ],
        out_specs=pl.BlockSpec((BB, LATENT), lambda i: (i, 0)),
        out_shape=jax.ShapeDtypeStruct((B, LATENT), jnp.float32),
    )(te, ce, W1, W2, b2d)


def _as_bf16(x_pk):
    B = x_pk.shape[0]
    return jax.lax.bitcast_convert_type(x_pk, jnp.bfloat16).reshape(B, LATENT)


def kernel(task_id, cancer_id, task_table, cancer_table, W, b):
    te_pk, ce_pk = _sc_gather(task_table, cancer_table, task_id, cancer_id)
    W1 = W[:, :LATENT][:, _PAIR_PERM].astype(jnp.bfloat16)
    W2 = W[:, LATENT:][:, _PAIR_PERM].astype(jnp.bfloat16)
    return _tc_project(
        _as_bf16(te_pk), _as_bf16(ce_pk), W1, W2, b.reshape(1, LATENT)
    )


# speedup vs baseline: 2.8996x; 2.8996x over previous
import functools

import jax
import jax.numpy as jnp
from jax import lax
from jax.experimental import pallas as pl
from jax.experimental.pallas import tpu as pltpu
from jax.experimental.pallas import tpu_sc as plsc

LATENT = 128
IDX_CHUNK = 128


@functools.partial(jax.jit, static_argnums=())
def _sc_gather(task_table, cancer_table, task_id, cancer_id):
    B = task_id.shape[0]
    D = task_table.shape[1]
    info = plsc.get_sparse_core_info()
    nw = info.num_cores * info.num_subcores
    b_per_w = B // nw
    n_chunk = b_per_w // IDX_CHUNK

    n_total = 2 * n_chunk
    NBUF = 3

    mesh = plsc.VectorSubcoreMesh(core_axis_name="c", subcore_axis_name="s")

    @functools.partial(
        pl.kernel,
        mesh=mesh,
        out_type=[
            jax.ShapeDtypeStruct((B, D), jnp.float32),
            jax.ShapeDtypeStruct((B, D), jnp.float32),
        ],
        scratch_types=[
            pltpu.VMEM((b_per_w,), jnp.int32),
            pltpu.VMEM((b_per_w,), jnp.int32),
            pltpu.VMEM((NBUF, IDX_CHUNK, D), jnp.float32),
            pltpu.SemaphoreType.DMA((NBUF,)),
            pltpu.SemaphoreType.DMA((NBUF,)),
            pltpu.SemaphoreType.DMA,
        ],
    )
    def gather2(t_tab, c_tab, t_idx, c_idx, t_out, c_out, tid_v, cid_v, rows_v,
                sem_g, sem_o, sem_i):
        wid = lax.axis_index("s") * info.num_cores + lax.axis_index("c")
        base = wid * b_per_w
        icp1 = pltpu.async_copy(t_idx.at[pl.ds(base, b_per_w)], tid_v, sem_i)
        icp2 = pltpu.async_copy(c_idx.at[pl.ds(base, b_per_w)], cid_v, sem_i)
        icp1.wait()
        icp2.wait()

        def start_gather(c):
            tab = t_tab if c < n_chunk else c_tab
            idx_v = tid_v if c < n_chunk else cid_v
            j = c % n_chunk
            return pltpu.async_copy(
                tab.at[idx_v.at[pl.ds(j * IDX_CHUNK, IDX_CHUNK)]],
                rows_v.at[c % NBUF],
                sem_g.at[c % NBUF],
            )

        def start_out(c):
            out = t_out if c < n_chunk else c_out
            j = c % n_chunk
            return pltpu.async_copy(
                rows_v.at[c % NBUF],
                out.at[pl.ds(base + j * IDX_CHUNK, IDX_CHUNK)],
                sem_o.at[c % NBUF],
            )

        gcp = [None] * n_total
        ocp = [None] * n_total
        gcp[0] = start_gather(0)
        for c in range(n_total):
            if c + 1 < n_total:
                if c + 1 >= NBUF:
                    ocp[c + 1 - NBUF].wait()
                gcp[c + 1] = start_gather(c + 1)
            gcp[c].wait()
            ocp[c] = start_out(c)
        for c in range(n_total - NBUF, n_total):
            ocp[c].wait()

    return gather2(task_table, cancer_table, task_id, cancer_id)


def _tc_project(te, ce, W1, W2, b2d):
    B = te.shape[0]
    BB = 8192

    def body(te_ref, ce_ref, w1_ref, w2_ref, b_ref, o_ref):
        acc = lax.dot_general(
            te_ref[...], w1_ref[...], (((1,), (1,)), ((), ())),
            preferred_element_type=jnp.float32,
        )
        acc += lax.dot_general(
            ce_ref[...], w2_ref[...], (((1,), (1,)), ((), ())),
            preferred_element_type=jnp.float32,
        )
        o_ref[...] = acc + b_ref[...]

    return pl.pallas_call(
        body,
        grid=(B // BB,),
        in_specs=[
            pl.BlockSpec((BB, LATENT), lambda i: (i, 0)),
            pl.BlockSpec((BB, LATENT), lambda i: (i, 0)),
            pl.BlockSpec((LATENT, LATENT), lambda i: (0, 0)),
            pl.BlockSpec((LATENT, LATENT), lambda i: (0, 1)),
            pl.BlockSpec((1, LATENT), lambda i: (0, 0)),
        ],
        out_specs=pl.BlockSpec((BB, LATENT), lambda i: (i, 0)),
        out_shape=jax.ShapeDtypeStruct((B, LATENT), jnp.float32),
    )(te, ce, W1, W2, b2d)


def kernel(task_id, cancer_id, task_table, cancer_table, W, b):
    te, ce = _sc_gather(task_table, cancer_table, task_id, cancer_id)
    return _tc_project(te, ce, W, W, b.reshape(1, LATENT))
